# baseline (speedup 1.0000x reference)
import functools

import jax
import jax.numpy as jnp
from jax import lax
from jax.experimental import pallas as pl
from jax.experimental.pallas import tpu as pltpu
from jax.experimental.pallas import tpu_sc as plsc

NC = 2
NS = 16
NW = NC * NS
CHUNK = 64


def _mm1_body(x_ref, w_ref, o_ref):
    o_ref[...] = jnp.dot(x_ref[...], w_ref[...],
                         preferred_element_type=jnp.float32
                         ).astype(o_ref.dtype)


def _mm2_body(p0_ref, p1_ref, w_ref, o_ref):
    h = jnp.maximum(p0_ref[...] + p1_ref[...], 0.0)
    o_ref[...] = jnp.dot(h, w_ref[...], preferred_element_type=jnp.float32
                         ).astype(o_ref.dtype)


def _softmax_body(p0_ref, p1_ref, o_ref):
    z = p0_ref[...] + p1_ref[...]
    z = z - jnp.max(z, axis=-1, keepdims=True)
    e = jnp.exp(z)
    o_ref[...] = e / jnp.sum(e, axis=-1, keepdims=True)


def _mm1(x, w, bm):
    n, c = x.shape
    h = w.shape[1]
    return pl.pallas_call(
        _mm1_body,
        grid=(n // bm,),
        in_specs=[pl.BlockSpec((bm, c), lambda i: (i, 0)),
                  pl.BlockSpec((c, h), lambda i: (0, 0))],
        out_specs=pl.BlockSpec((bm, h), lambda i: (i, 0)),
        out_shape=jax.ShapeDtypeStruct((n, h), jnp.bfloat16),
    )(x, w)


def _mm2(p0, p1, w, bm):
    n, h = p0.shape
    f = w.shape[1]
    return pl.pallas_call(
        _mm2_body,
        grid=(n // bm,),
        in_specs=[pl.BlockSpec((bm, h), lambda i: (i, 0)),
                  pl.BlockSpec((bm, h), lambda i: (i, 0)),
                  pl.BlockSpec((h, f), lambda i: (0, 0))],
        out_specs=pl.BlockSpec((bm, f), lambda i: (i, 0)),
        out_shape=jax.ShapeDtypeStruct((n, f), jnp.bfloat16),
    )(p0, p1, w)


def _softmax(p0, p1, bm):
    n, f = p0.shape
    return pl.pallas_call(
        _softmax_body,
        grid=(n // bm,),
        in_specs=[pl.BlockSpec((bm, f), lambda i: (i, 0)),
                  pl.BlockSpec((bm, f), lambda i: (i, 0))],
        out_specs=pl.BlockSpec((bm, f), lambda i: (i, 0)),
        out_shape=jax.ShapeDtypeStruct((n, f), jnp.float32),
    )(p0, p1)


def _make_spmm(n, d, ep):
    epw = ep // NW
    nch = epw // CHUNK
    rpt = n // NS
    mesh = plsc.VectorSubcoreMesh(core_axis_name="c", subcore_axis_name="s",
                                  num_cores=NC, num_subcores=NS)

    @functools.partial(
        pl.kernel,
        mesh=mesh,
        compiler_params=pltpu.CompilerParams(use_tc_tiling_on_sc=False,
                                             needs_layout_passes=False),
        out_type=jax.ShapeDtypeStruct((NC, n, d), jnp.float32),
        scratch_types=[
            pltpu.VMEM((nch, CHUNK), jnp.int32),
            pltpu.VMEM((nch, CHUNK), jnp.bfloat16),
            pltpu.VMEM((3, CHUNK), jnp.int32),
            pltpu.VMEM((8, CHUNK), jnp.int32),
            pltpu.VMEM((CHUNK, d), jnp.bfloat16),
            pltpu.VMEM((CHUNK, d), jnp.bfloat16),
            pltpu.VMEM((CHUNK, d), jnp.bfloat16),
            pltpu.VMEM((CHUNK, d), jnp.float32),
            pltpu.VMEM((CHUNK, d), jnp.float32),
            pltpu.VMEM_SHARED((n, d), jnp.float32),
            [pltpu.SemaphoreType.DMA] * 3,
            [pltpu.SemaphoreType.DMA] * 2,
        ],
    )
    def spmm(y_hbm, packed_hbm, vals_hbm, zeros_hbm, out_hbm,
             pidx, vvals, cidx, ridx, gbuf0, gbuf1, gbuf2, sbuf0, sbuf1,
             acc, gsem, ssem):
        gbufs = (gbuf0, gbuf1, gbuf2)
        sbufs = (sbuf0, sbuf1)
        cid = lax.axis_index("c")
        sid = lax.axis_index("s")
        wid = cid * NS + sid
        r0 = sid * rpt
        pltpu.sync_copy(zeros_hbm.at[pl.ds(r0, rpt)], acc.at[pl.ds(r0, rpt)])
        pltpu.sync_copy(packed_hbm.at[wid], pidx)
        pltpu.sync_copy(vals_hbm.at[wid], vvals)
        plsc.subcore_barrier()

        def unpack(j, k3):
            m = lax.rem(j, 8)
            for g in range(CHUNK // 16):
                sl = pl.ds(g * 16, 16)
                x = pidx[j, sl]
                cidx[k3, sl] = lax.bitwise_and(x, jnp.int32(0xFFFF))
                ridx[m, sl] = lax.shift_right_logical(x, jnp.int32(16))

        def scale(j, k3, k2):
            def group_body(g, gcarry):
                vg = vvals[j, pl.ds(g * 32, 32)]
                va, vb = plsc.unpack(vg, format=plsc.PackFormat.INTERLEAVED,
                                     preferred_element_type=jnp.float32)
                for jj in range(32):
                    e = g * 32 + jj
                    src = va if jj % 2 == 0 else vb
                    bv = lax.gather(
                        src, jnp.full((16, 1), jj // 2, jnp.int32),
                        lax.GatherDimensionNumbers(
                            offset_dims=(), collapsed_slice_dims=(0,),
                            start_index_map=(0,)),
                        slice_sizes=(1,),
                        mode=lax.GatherScatterMode.PROMISE_IN_BOUNDS)
                    for dd in range(d // 32):
                        xb = gbufs[k3][e, pl.ds(dd * 32, 32)]
                        a, b = plsc.unpack(
                            xb, format=plsc.PackFormat.INTERLEAVED,
                            preferred_element_type=jnp.float32)
                        sbufs[k2][e, pl.ds(dd * 32, 16)] = a * bv
                        sbufs[k2][e, pl.ds(dd * 32 + 16, 16)] = b * bv
                return gcarry

            lax.fori_loop(0, CHUNK // 32, group_body, 0)

        def start_gather(k3):
            pltpu.async_copy(y_hbm.at[cidx.at[k3]], gbufs[k3], gsem[k3])

        def wait_gather(k3):
            pltpu.make_async_copy(y_hbm.at[cidx.at[k3]], gbufs[k3],
                                  gsem[k3]).wait()

        def start_scatter(j, k2):
            m = lax.rem(j, 8)
            pltpu.async_copy(sbufs[k2], acc.at[ridx.at[m]], ssem[k2],
                             add=True)

        def wait_scatter(j, k2):
            m = lax.rem(j, 8)
            pltpu.make_async_copy(sbufs[k2], acc.at[ridx.at[m]],
                                  ssem[k2]).wait()

        for i in range(3):
            unpack(i, i)
            start_gather(i)

        def six_body(t, carry):
            for k in range(6):
                j = 6 * t + k
                k3 = k % 3
                k2 = k % 2
                wait_gather(k3)

                @pl.when(j >= 2)
                def _():
                    wait_scatter(j - 2, k2)

                scale(j, k3, k2)

                @pl.when(j + 3 < nch)
                def _():
                    unpack(j + 3, k3)
                    start_gather(k3)

                start_scatter(j, k2)
            return carry

        lax.fori_loop(0, nch // 6, six_body, 0)
        wait_scatter(nch - 2, 0)
        wait_scatter(nch - 1, 1)
        plsc.subcore_barrier()
        pltpu.sync_copy(acc.at[pl.ds(r0, rpt)],
                        out_hbm.at[cid, pl.ds(r0, rpt)])

    return spmm


def _interleave_perm(d):
    return [32 * (j // 32) + (j % 32) // 2 + (16 if j % 2 else 0)
            for j in range(d)]


def kernel(X, A_rows, A_cols, A_vals, W0, W1):
    n, c = X.shape
    h = W0.shape[1]
    f = W1.shape[1]
    e = A_rows.shape[0]

    grain = NW * CHUNK * 6
    ep = ((e + grain - 1) // grain) * grain
    pad = ep - e
    nch = ep // (NW * CHUNK)
    packed = jnp.concatenate(
        [(A_rows << 16) | A_cols,
         jnp.zeros((pad,), jnp.int32)]).reshape(NW, nch, CHUNK)
    vals_p = jnp.concatenate(
        [A_vals, jnp.zeros((pad,), jnp.float32)]
    ).astype(jnp.bfloat16).reshape(NW, nch, CHUNK)

    rgrain = NS * 8
    np_ = ((n + rgrain - 1) // rgrain) * rgrain
    zeros_h = jnp.zeros((np_, h), jnp.float32)
    zeros_f = jnp.zeros((np_, f), jnp.float32)

    W0p = W0[:, jnp.array(_interleave_perm(h))]
    W1p = W1[:, jnp.array(_interleave_perm(f))]

    bm = 1000
    y0 = _mm1(X, W0p, bm)
    p1 = _make_spmm(np_, h, ep)(y0, packed, vals_p, zeros_h)
    y1 = _mm2(p1[0, :n], p1[1, :n], W1p, bm)
    p2 = _make_spmm(np_, f, ep)(y1, packed, vals_p, zeros_f)
    return _softmax(p2[0, :n], p2[1, :n], bm)


# speedup vs baseline: 1.2953x; 1.2953x over previous
import functools

import jax
import jax.numpy as jnp
from jax import lax
from jax.experimental import pallas as pl
from jax.experimental.pallas import tpu as pltpu
from jax.experimental.pallas import tpu_sc as plsc

NC = 2
NS = 16
NW = NC * NS
CHUNK = 64


def _mm1_body(x_ref, w_ref, o_ref):
    o_ref[...] = jnp.dot(x_ref[...], w_ref[...],
                         preferred_element_type=jnp.float32
                         ).astype(o_ref.dtype)


def _mm2_body(p0_ref, p1_ref, w_ref, o_ref):
    h = jnp.maximum(p0_ref[...] + p1_ref[...], 0.0)
    o_ref[...] = jnp.dot(h, w_ref[...], preferred_element_type=jnp.float32
                         ).astype(o_ref.dtype)


def _softmax_body(p0_ref, p1_ref, o_ref):
    z = p0_ref[...] + p1_ref[...]
    z = z - jnp.max(z, axis=-1, keepdims=True)
    e = jnp.exp(z)
    o_ref[...] = e / jnp.sum(e, axis=-1, keepdims=True)


def _mm1(x, w, bm):
    n, c = x.shape
    h = w.shape[1]
    return pl.pallas_call(
        _mm1_body,
        grid=(n // bm,),
        in_specs=[pl.BlockSpec((bm, c), lambda i: (i, 0)),
                  pl.BlockSpec((c, h), lambda i: (0, 0))],
        out_specs=pl.BlockSpec((bm, h), lambda i: (i, 0)),
        out_shape=jax.ShapeDtypeStruct((n, h), jnp.bfloat16),
    )(x, w)


def _mm2(p0, p1, w, bm):
    n, h = p0.shape
    f = w.shape[1]
    return pl.pallas_call(
        _mm2_body,
        grid=(n // bm,),
        in_specs=[pl.BlockSpec((bm, h), lambda i: (i, 0)),
                  pl.BlockSpec((bm, h), lambda i: (i, 0)),
                  pl.BlockSpec((h, f), lambda i: (0, 0))],
        out_specs=pl.BlockSpec((bm, f), lambda i: (i, 0)),
        out_shape=jax.ShapeDtypeStruct((n, f), jnp.bfloat16),
    )(p0, p1, w)


def _softmax(p0, p1, bm):
    n, f = p0.shape
    return pl.pallas_call(
        _softmax_body,
        grid=(n // bm,),
        in_specs=[pl.BlockSpec((bm, f), lambda i: (i, 0)),
                  pl.BlockSpec((bm, f), lambda i: (i, 0))],
        out_specs=pl.BlockSpec((bm, f), lambda i: (i, 0)),
        out_shape=jax.ShapeDtypeStruct((n, f), jnp.float32),
    )(p0, p1)


def _make_spmm(n, d, ep):
    epw = ep // NW
    nch = epw // CHUNK
    rpt = n // NS
    mesh = plsc.VectorSubcoreMesh(core_axis_name="c", subcore_axis_name="s",
                                  num_cores=NC, num_subcores=NS)

    @functools.partial(
        pl.kernel,
        mesh=mesh,
        compiler_params=pltpu.CompilerParams(use_tc_tiling_on_sc=False,
                                             needs_layout_passes=False),
        out_type=jax.ShapeDtypeStruct((NC, n, d), jnp.float32),
        scratch_types=[
            pltpu.VMEM((nch, CHUNK), jnp.int32),
            pltpu.VMEM((nch, CHUNK), jnp.float32),
            pltpu.VMEM((2, CHUNK), jnp.int32),
            pltpu.VMEM((4, CHUNK), jnp.int32),
            pltpu.VMEM((CHUNK, d), jnp.bfloat16),
            pltpu.VMEM((CHUNK, d), jnp.bfloat16),
            pltpu.VMEM((CHUNK, d), jnp.float32),
            pltpu.VMEM((CHUNK, d), jnp.float32),
            pltpu.VMEM_SHARED((n, d), jnp.float32),
            [pltpu.SemaphoreType.DMA] * 2,
            [pltpu.SemaphoreType.DMA] * 2,
        ],
    )
    def spmm(y_hbm, packed_hbm, vals_hbm, zeros_hbm, out_hbm,
             pidx, vvals, cidx, ridx, gbuf0, gbuf1, sbuf0, sbuf1,
             acc, gsem, ssem):
        gbufs = (gbuf0, gbuf1)
        sbufs = (sbuf0, sbuf1)
        cid = lax.axis_index("c")
        sid = lax.axis_index("s")
        wid = cid * NS + sid
        r0 = sid * rpt
        pltpu.sync_copy(zeros_hbm.at[pl.ds(r0, rpt)], acc.at[pl.ds(r0, rpt)])
        pltpu.sync_copy(packed_hbm.at[wid], pidx)
        pltpu.sync_copy(vals_hbm.at[wid], vvals)
        plsc.subcore_barrier()

        def unpack(j, k):
            m = lax.rem(j, 4)
            for g in range(CHUNK // 16):
                sl = pl.ds(g * 16, 16)
                x = pidx[j, sl]
                cidx[k, sl] = lax.bitwise_and(x, jnp.int32(0xFFFF))
                ridx[m, sl] = lax.shift_right_logical(x, jnp.int32(16))

        def scale(j, k):
            def group_body(g, gcarry):
                vg = vvals[j, pl.ds(g * 16, 16)]
                for jj in range(16):
                    e = g * 16 + jj
                    bv = lax.gather(
                        vg, jnp.full((16, 1), jj, jnp.int32),
                        lax.GatherDimensionNumbers(
                            offset_dims=(), collapsed_slice_dims=(0,),
                            start_index_map=(0,)),
                        slice_sizes=(1,),
                        mode=lax.GatherScatterMode.PROMISE_IN_BOUNDS)
                    for dd in range(d // 32):
                        xb = gbufs[k][e, pl.ds(dd * 32, 32)]
                        a, b = plsc.unpack(
                            xb, format=plsc.PackFormat.INTERLEAVED,
                            preferred_element_type=jnp.float32)
                        sbufs[k][e, pl.ds(dd * 32, 16)] = a * bv
                        sbufs[k][e, pl.ds(dd * 32 + 16, 16)] = b * bv
                return gcarry

            lax.fori_loop(0, CHUNK // 16, group_body, 0)

        def start_gather(k):
            pltpu.async_copy(y_hbm.at[cidx.at[k]], gbufs[k], gsem[k])

        def wait_gather(k):
            pltpu.make_async_copy(y_hbm.at[cidx.at[k]], gbufs[k],
                                  gsem[k]).wait()

        def start_scatter(j, k):
            m = lax.rem(j, 4)
            pltpu.async_copy(sbufs[k], acc.at[ridx.at[m]], ssem[k],
                             add=True)

        def wait_scatter(j, k):
            m = lax.rem(j, 4)
            pltpu.make_async_copy(sbufs[k], acc.at[ridx.at[m]],
                                  ssem[k]).wait()

        unpack(0, 0)
        start_gather(0)
        unpack(1, 1)
        start_gather(1)

        def pair_body(p, carry):
            for k in range(2):
                j = 2 * p + k
                wait_gather(k)

                @pl.when(j >= 2)
                def _():
                    wait_scatter(j - 2, k)

                scale(j, k)

                @pl.when(j + 2 < nch)
                def _():
                    unpack(j + 2, k)
                    start_gather(k)

                start_scatter(j, k)
            return carry

        lax.fori_loop(0, nch // 2, pair_body, 0)
        wait_scatter(nch - 2, 0)
        wait_scatter(nch - 1, 1)
        plsc.subcore_barrier()
        pltpu.sync_copy(acc.at[pl.ds(r0, rpt)],
                        out_hbm.at[cid, pl.ds(r0, rpt)])

    return spmm


def _interleave_perm(d):
    return [32 * (j // 32) + (j % 32) // 2 + (16 if j % 2 else 0)
            for j in range(d)]


def kernel(X, A_rows, A_cols, A_vals, W0, W1):
    n, c = X.shape
    h = W0.shape[1]
    f = W1.shape[1]
    e = A_rows.shape[0]

    grain = NW * CHUNK * 2
    ep = ((e + grain - 1) // grain) * grain
    pad = ep - e
    nch = ep // (NW * CHUNK)
    packed = jnp.concatenate(
        [(A_rows << 16) | A_cols,
         jnp.zeros((pad,), jnp.int32)]).reshape(NW, nch, CHUNK)
    vals_p = jnp.concatenate(
        [A_vals, jnp.zeros((pad,), jnp.float32)]).reshape(NW, nch, CHUNK)

    rgrain = NS * 8
    np_ = ((n + rgrain - 1) // rgrain) * rgrain
    zeros_h = jnp.zeros((np_, h), jnp.float32)
    zeros_f = jnp.zeros((np_, f), jnp.float32)

    W0p = W0[:, jnp.array(_interleave_perm(h))]
    W1p = W1[:, jnp.array(_interleave_perm(f))]

    bm = 1000
    y0 = _mm1(X, W0p, bm)
    p1 = _make_spmm(np_, h, ep)(y0, packed, vals_p, zeros_h)
    y1 = _mm2(p1[0, :n], p1[1, :n], W1p, bm)
    p2 = _make_spmm(np_, f, ep)(y1, packed, vals_p, zeros_f)
    return _softmax(p2[0, :n], p2[1, :n], bm)


# speedup vs baseline: 1.3056x; 1.0079x over previous
import functools

import jax
import jax.numpy as jnp
from jax import lax
from jax.experimental import pallas as pl
from jax.experimental.pallas import tpu as pltpu
from jax.experimental.pallas import tpu_sc as plsc

NC = 2
NS = 16
NW = NC * NS
CHUNK = 64


def _pack_rows(r):
    ui = lax.bitcast_convert_type(r, jnp.int32)
    t = lax.shift_right_logical(
        ui + jnp.int32(0x7FFF)
        + lax.bitwise_and(lax.shift_right_logical(ui, jnp.int32(16)),
                          jnp.int32(1)),
        jnp.int32(16))
    half = r.shape[1] // 2
    return lax.bitwise_or(t[:, :half],
                          lax.shift_left(t[:, half:], jnp.int32(16)))


def _mm1_body(x_ref, w_ref, o_ref):
    o_ref[...] = _pack_rows(jnp.dot(x_ref[...], w_ref[...],
                                    preferred_element_type=jnp.float32))


def _mm2_body(p0_ref, p1_ref, w_ref, o_ref):
    h = jnp.maximum(p0_ref[...] + p1_ref[...], 0.0)
    o_ref[...] = _pack_rows(jnp.dot(h, w_ref[...],
                                    preferred_element_type=jnp.float32))


def _softmax_body(p0_ref, p1_ref, o_ref):
    z = p0_ref[...] + p1_ref[...]
    z = z - jnp.max(z, axis=-1, keepdims=True)
    e = jnp.exp(z)
    o_ref[...] = e / jnp.sum(e, axis=-1, keepdims=True)


def _mm1(x, w, bm):
    n, c = x.shape
    h = w.shape[1]
    return pl.pallas_call(
        _mm1_body,
        grid=(n // bm,),
        in_specs=[pl.BlockSpec((bm, c), lambda i: (i, 0)),
                  pl.BlockSpec((c, h), lambda i: (0, 0))],
        out_specs=pl.BlockSpec((bm, h // 2), lambda i: (i, 0)),
        out_shape=jax.ShapeDtypeStruct((n, h // 2), jnp.int32),
    )(x, w)


def _mm2(p0, p1, w, bm):
    n, h = p0.shape
    f = w.shape[1]
    return pl.pallas_call(
        _mm2_body,
        grid=(n // bm,),
        in_specs=[pl.BlockSpec((bm, h), lambda i: (i, 0)),
                  pl.BlockSpec((bm, h), lambda i: (i, 0)),
                  pl.BlockSpec((h, f), lambda i: (0, 0))],
        out_specs=pl.BlockSpec((bm, f // 2), lambda i: (i, 0)),
        out_shape=jax.ShapeDtypeStruct((n, f // 2), jnp.int32),
    )(p0, p1, w)


def _softmax(p0, p1, bm):
    n, f = p0.shape
    return pl.pallas_call(
        _softmax_body,
        grid=(n // bm,),
        in_specs=[pl.BlockSpec((bm, f), lambda i: (i, 0)),
                  pl.BlockSpec((bm, f), lambda i: (i, 0))],
        out_specs=pl.BlockSpec((bm, f), lambda i: (i, 0)),
        out_shape=jax.ShapeDtypeStruct((n, f), jnp.float32),
    )(p0, p1)


def _make_spmm(n, d, ep):
    epw = ep // NW
    nch = epw // CHUNK
    rpt = n // NS
    mesh = plsc.VectorSubcoreMesh(core_axis_name="c", subcore_axis_name="s",
                                  num_cores=NC, num_subcores=NS)

    @functools.partial(
        pl.kernel,
        mesh=mesh,
        compiler_params=pltpu.CompilerParams(use_tc_tiling_on_sc=False,
                                             needs_layout_passes=False),
        out_type=jax.ShapeDtypeStruct((NC, n, d), jnp.float32),
        scratch_types=[
            pltpu.VMEM((nch, CHUNK), jnp.int32),
            pltpu.VMEM((nch, CHUNK), jnp.float32),
            pltpu.VMEM((2, CHUNK), jnp.int32),
            pltpu.VMEM((4, CHUNK), jnp.int32),
            pltpu.VMEM((CHUNK, d // 2), jnp.int32),
            pltpu.VMEM((CHUNK, d // 2), jnp.int32),
            pltpu.VMEM((CHUNK, d), jnp.float32),
            pltpu.VMEM((CHUNK, d), jnp.float32),
            pltpu.VMEM_SHARED((n, d), jnp.float32),
            [pltpu.SemaphoreType.DMA] * 2,
            [pltpu.SemaphoreType.DMA] * 2,
        ],
    )
    def spmm(y_hbm, packed_hbm, vals_hbm, zeros_hbm, out_hbm,
             pidx, vvals, cidx, ridx, gbuf0, gbuf1, sbuf0, sbuf1,
             acc, gsem, ssem):
        gbufs = (gbuf0, gbuf1)
        sbufs = (sbuf0, sbuf1)
        cid = lax.axis_index("c")
        sid = lax.axis_index("s")
        wid = cid * NS + sid
        r0 = sid * rpt
        pltpu.sync_copy(zeros_hbm.at[pl.ds(r0, rpt)], acc.at[pl.ds(r0, rpt)])
        pltpu.sync_copy(packed_hbm.at[wid], pidx)
        pltpu.sync_copy(vals_hbm.at[wid], vvals)
        plsc.subcore_barrier()

        def unpack(j, k):
            m = lax.rem(j, 4)
            for g in range(CHUNK // 16):
                sl = pl.ds(g * 16, 16)
                x = pidx[j, sl]
                cidx[k, sl] = lax.bitwise_and(x, jnp.int32(0xFFFF))
                ridx[m, sl] = lax.shift_right_logical(x, jnp.int32(16))

        def scale(j, k):
            def group_body(g, gcarry):
                vg = vvals[j, pl.ds(g * 16, 16)]
                for jj in range(16):
                    e = g * 16 + jj
                    bv = lax.gather(
                        vg, jnp.full((16, 1), jj, jnp.int32),
                        lax.GatherDimensionNumbers(
                            offset_dims=(), collapsed_slice_dims=(0,),
                            start_index_map=(0,)),
                        slice_sizes=(1,),
                        mode=lax.GatherScatterMode.PROMISE_IN_BOUNDS)
                    for dd in range(d // 32):
                        x = gbufs[k][e, pl.ds(dd * 16, 16)]
                        a = plsc.bitcast(
                            lax.shift_left(x, jnp.int32(16)), jnp.float32)
                        b = plsc.bitcast(
                            lax.bitwise_and(x, jnp.int32(-65536)),
                            jnp.float32)
                        sbufs[k][e, pl.ds(dd * 32, 16)] = a * bv
                        sbufs[k][e, pl.ds(dd * 32 + 16, 16)] = b * bv
                return gcarry

            lax.fori_loop(0, CHUNK // 16, group_body, 0)

        def start_gather(k):
            pltpu.async_copy(y_hbm.at[cidx.at[k]], gbufs[k], gsem[k])

        def wait_gather(k):
            pltpu.make_async_copy(y_hbm.at[cidx.at[k]], gbufs[k],
                                  gsem[k]).wait()

        def start_scatter(j, k):
            m = lax.rem(j, 4)
            pltpu.async_copy(sbufs[k], acc.at[ridx.at[m]], ssem[k],
                             add=True)

        def wait_scatter(j, k):
            m = lax.rem(j, 4)
            pltpu.make_async_copy(sbufs[k], acc.at[ridx.at[m]],
                                  ssem[k]).wait()

        unpack(0, 0)
        start_gather(0)
        unpack(1, 1)
        start_gather(1)

        def pair_body(p, carry):
            for k in range(2):
                j = 2 * p + k
                wait_gather(k)

                @pl.when(j >= 2)
                def _():
                    wait_scatter(j - 2, k)

                scale(j, k)

                @pl.when(j + 2 < nch)
                def _():
                    unpack(j + 2, k)
                    start_gather(k)

                start_scatter(j, k)
            return carry

        lax.fori_loop(0, nch // 2, pair_body, 0)
        wait_scatter(nch - 2, 0)
        wait_scatter(nch - 1, 1)
        plsc.subcore_barrier()
        pltpu.sync_copy(acc.at[pl.ds(r0, rpt)],
                        out_hbm.at[cid, pl.ds(r0, rpt)])

    return spmm


def _interleave_perm(d):
    lo = [32 * (m // 16) + m % 16 for m in range(d // 2)]
    hi = [32 * (m // 16) + 16 + m % 16 for m in range(d // 2)]
    return lo + hi


def kernel(X, A_rows, A_cols, A_vals, W0, W1):
    n, c = X.shape
    h = W0.shape[1]
    f = W1.shape[1]
    e = A_rows.shape[0]

    grain = NW * CHUNK * 2
    ep = ((e + grain - 1) // grain) * grain
    pad = ep - e
    nch = ep // (NW * CHUNK)
    packed = jnp.concatenate(
        [(A_rows << 16) | A_cols,
         jnp.zeros((pad,), jnp.int32)]).reshape(NW, nch, CHUNK)
    vals_p = jnp.concatenate(
        [A_vals, jnp.zeros((pad,), jnp.float32)]).reshape(NW, nch, CHUNK)

    rgrain = NS * 8
    np_ = ((n + rgrain - 1) // rgrain) * rgrain
    zeros_h = jnp.zeros((np_, h), jnp.float32)
    zeros_f = jnp.zeros((np_, f), jnp.float32)

    W0p = W0[:, jnp.array(_interleave_perm(h))]
    W1p = W1[:, jnp.array(_interleave_perm(f))]

    bm = 1000
    y0 = _mm1(X, W0p, bm)
    p1 = _make_spmm(np_, h, ep)(y0, packed, vals_p, zeros_h)
    y1 = _mm2(p1[0, :n], p1[1, :n], W1p, bm)
    p2 = _make_spmm(np_, f, ep)(y1, packed, vals_p, zeros_f)
    return _softmax(p2[0, :n], p2[1, :n], bm)
